# Initial kernel scaffold; baseline (speedup 1.0000x reference)
import functools

import jax
import jax.numpy as jnp
from jax import lax
from jax.experimental import pallas as pl
from jax.experimental.pallas import tpu as pltpu
from jax.experimental.pallas import tpu_sc as plsc

_NOBJ = 50
_NSLOT = 128


def _sc_match(boxes_p, cls_p, prior_p):
    B = boxes_p.shape[0]
    mesh = plsc.VectorSubcoreMesh(core_axis_name="c", subcore_axis_name="s")
    out_type = (
        jax.ShapeDtypeStruct((B, 1, _NSLOT), jnp.int32),
        jax.ShapeDtypeStruct((B, 1, _NSLOT), jnp.float32),
        jax.ShapeDtypeStruct((B, 1, _NSLOT), jnp.float32),
    )
    scratch = [
        pltpu.VMEM((56, 4), jnp.int32),
        pltpu.VMEM((64,), jnp.int32),
        pltpu.VMEM((104, 2), jnp.float32),
        pltpu.VMEM((_NSLOT,), jnp.int32),
        pltpu.VMEM((_NSLOT,), jnp.float32),
        pltpu.VMEM((_NSLOT,), jnp.float32),
    ]

    @functools.partial(pl.kernel, out_type=out_type, mesh=mesh,
                       scratch_types=scratch)
    def k(boxes_hbm, cls_hbm, prior_hbm, ocls, oox, ooy,
          box_v, cls_v, prior_v, gcls_v, gox_v, goy_v):
        wid = lax.axis_index("s") * 2 + lax.axis_index("c")

        @pl.when(wid < B)
        def _():
            b = wid
            pltpu.sync_copy(boxes_hbm.at[b], box_v)
            pltpu.sync_copy(cls_hbm.at[b], cls_v)
            pltpu.sync_copy(prior_hbm, prior_v)

            iota = lax.broadcasted_iota(jnp.int32, (16,), 0)
            zero_i = jnp.zeros((16,), jnp.int32)
            zero_f = jnp.zeros((16,), jnp.float32)
            for i in range(_NSLOT // 16):
                gcls_v[pl.ds(i * 16, 16)] = zero_i
                gox_v[pl.ds(i * 16, 16)] = zero_f
                goy_v[pl.ds(i * 16, 16)] = zero_f

            ks, cs, oxs, oys = [], [], [], []
            for c in range(4):
                ridx = jnp.minimum(iota + c * 16, _NOBJ - 1)
                x0 = plsc.load_gather(box_v, [ridx, zero_i])
                y0 = plsc.load_gather(box_v, [ridx, jnp.full((16,), 1, jnp.int32)])
                x1 = plsc.load_gather(box_v, [ridx, jnp.full((16,), 2, jnp.int32)])
                y1 = plsc.load_gather(box_v, [ridx, jnp.full((16,), 3, jnp.int32)])
                cx = lax.shift_right_arithmetic(x0 + x1, 1)
                cy = lax.shift_right_arithmetic(y0 + y1, 1)
                kc = (lax.shift_right_arithmetic(cy, 5) * 10
                      + lax.shift_right_arithmetic(cx, 5))
                px = plsc.load_gather(prior_v, [kc, zero_i])
                py = plsc.load_gather(prior_v, [kc, jnp.full((16,), 1, jnp.int32)])
                ks.append(kc)
                oxs.append(cx.astype(jnp.float32) - px)
                oys.append(cy.astype(jnp.float32) - py)
                cs.append(cls_v[pl.ds(c * 16, 16)])

            for n in range(_NOBJ):
                c, l = divmod(n, 16)
                m = iota == l
                plsc.store_scatter(gcls_v, [ks[c]], cs[c], mask=m)
                plsc.store_scatter(gox_v, [ks[c]], oxs[c], mask=m)
                plsc.store_scatter(goy_v, [ks[c]], oys[c], mask=m)

            pltpu.sync_copy(gcls_v, ocls.at[b, 0])
            pltpu.sync_copy(gox_v, oox.at[b, 0])
            pltpu.sync_copy(goy_v, ooy.at[b, 0])

    return k(boxes_p, cls_p, prior_p)


def _tc_loss(pred_r, gcls, gox, goy, interpret=False):
    B, CC, P = pred_r.shape
    NC = CC - 2

    def body(pred_ref, cls_ref, ox_ref, oy_ref, loc_ref, conf_ref, acc_ref):
        b = pl.program_id(0)
        x = pred_ref[0]
        rowi = lax.broadcasted_iota(jnp.int32, (CC, 1), 0)
        e = jnp.exp(x)
        ecls = jnp.where(rowi < NC, e, 0.0)
        sumexp = jnp.sum(ecls, axis=0, keepdims=True)
        lse = jnp.log(sumexp)
        s = x[0:1, :] - lse

        col = lax.broadcasted_iota(jnp.int32, (1, P), 1)
        m1 = jnp.min(s)
        i1 = jnp.min(jnp.where(s == m1, col, P))
        s2 = jnp.where(col == i1, jnp.float32(jnp.inf), s)
        m2 = jnp.min(s2)
        i2 = jnp.min(jnp.where(s2 == m2, col, P))

        g = cls_ref[0, 0]
        g2 = cls_ref[0]
        pos = g2 > 0
        posf = jnp.where(pos, 1.0, 0.0)
        npos_b = jnp.sum(posf)

        xs = x[:, 0:_NSLOT]
        px = jnp.sum(jnp.where(rowi == NC, xs, 0.0), axis=0, keepdims=True)
        py = jnp.sum(jnp.where(rowi == NC + 1, xs, 0.0), axis=0, keepdims=True)
        gx = ox_ref[0]
        gy = oy_ref[0]
        loc_b = jnp.sum(((gx - px) ** 2 + (gy - py) ** 2) * posf)

        onehot = jnp.where(rowi == g2, 1.0, 0.0)
        picked = jnp.sum(xs * onehot, axis=0, keepdims=True)
        lse_s = lse[:, 0:_NSLOT]
        ce_b = jnp.sum((lse_s - picked) * posf)

        lane = lax.broadcasted_iota(jnp.int32, (1, _NSLOT), 1)
        isp1 = jnp.sum(jnp.where((lane == i1) & pos, 1.0, 0.0)) > 0.0
        isp2 = jnp.sum(jnp.where((lane == i2) & pos, 1.0, 0.0)) > 0.0
        ce_b = ce_b + jnp.where(isp1, 0.0, -m1) + jnp.where(isp2, 0.0, -m2)
        nsel_b = npos_b + jnp.where(isp1, 0.0, 1.0) + jnp.where(isp2, 0.0, 1.0)

        @pl.when(b == 0)
        def _():
            acc_ref[0] = loc_b
            acc_ref[1] = npos_b
            acc_ref[2] = ce_b
            acc_ref[3] = nsel_b

        @pl.when(b > 0)
        def _():
            acc_ref[0] += loc_b
            acc_ref[1] += npos_b
            acc_ref[2] += ce_b
            acc_ref[3] += nsel_b

        loc_ref[0, 0] = acc_ref[0] / (jnp.maximum(acc_ref[1], 1.0) * 2.0)
        conf_ref[0, 0] = acc_ref[2] / jnp.maximum(acc_ref[3], 1.0)

    return pl.pallas_call(
        body,
        grid=(B,),
        in_specs=[
            pl.BlockSpec((1, CC, P), lambda b: (b, 0, 0)),
            pl.BlockSpec((1, 1, _NSLOT), lambda b: (b, 0, 0)),
            pl.BlockSpec((1, 1, _NSLOT), lambda b: (b, 0, 0)),
            pl.BlockSpec((1, 1, _NSLOT), lambda b: (b, 0, 0)),
        ],
        out_specs=[
            pl.BlockSpec(memory_space=pltpu.SMEM),
            pl.BlockSpec(memory_space=pltpu.SMEM),
        ],
        out_shape=[
            jax.ShapeDtypeStruct((1, 1), jnp.float32),
            jax.ShapeDtypeStruct((1, 1), jnp.float32),
        ],
        scratch_shapes=[pltpu.SMEM((4,), jnp.float32)],
        interpret=interpret,
    )(pred_r, gcls, gox, goy)


def kernel(prediction_3d, boxes, classids, priorBox_2d):
    boxes_p = jnp.pad(boxes.astype(jnp.int32), ((0, 0), (0, 6), (0, 0)))
    cls_p = jnp.pad(classids.astype(jnp.int32), ((0, 0), (0, 14)))
    prior_p = priorBox_2d[:104]
    gcls, gox, goy = _sc_match(boxes_p, cls_p, prior_p)
    pred_r = jnp.concatenate(
        [prediction_3d[:, :, 2:], prediction_3d[:, :, :2]], axis=-1)
    pred_r = jnp.transpose(pred_r, (0, 2, 1))
    loc, conf = _tc_loss(pred_r, gcls, gox, goy)
    return (loc[0, 0], conf[0, 0])


# speedup vs baseline: 7.4229x; 7.4229x over previous
import functools

import jax
import jax.numpy as jnp
from jax import lax
from jax.experimental import pallas as pl
from jax.experimental.pallas import tpu as pltpu
from jax.experimental.pallas import tpu_sc as plsc

_NOBJ = 50
_NSLOT = 128


def _sc_match(boxes_p, cls_p, prior_p):
    B = boxes_p.shape[0]
    mesh = plsc.VectorSubcoreMesh(core_axis_name="c", subcore_axis_name="s")
    out_type = (
        jax.ShapeDtypeStruct((B, 1, _NSLOT), jnp.int32),
        jax.ShapeDtypeStruct((B, 1, _NSLOT), jnp.float32),
        jax.ShapeDtypeStruct((B, 1, _NSLOT), jnp.float32),
    )
    scratch = [
        pltpu.VMEM((224,), jnp.int32),
        pltpu.VMEM((64,), jnp.int32),
        pltpu.VMEM((208,), jnp.float32),
        pltpu.VMEM((_NSLOT,), jnp.int32),
        pltpu.VMEM((_NSLOT,), jnp.float32),
        pltpu.VMEM((_NSLOT,), jnp.float32),
    ]

    @functools.partial(
        pl.kernel, out_type=out_type, mesh=mesh, scratch_types=scratch,
        compiler_params=pltpu.CompilerParams(needs_layout_passes=False))
    def k(boxes_hbm, cls_hbm, prior_hbm, ocls, oox, ooy,
          box_v, cls_v, prior_v, gcls_v, gox_v, goy_v):
        wid = lax.axis_index("s") * 2 + lax.axis_index("c")

        @pl.when(wid < B)
        def _():
            b = wid
            pltpu.sync_copy(boxes_hbm.at[b], box_v)
            pltpu.sync_copy(cls_hbm.at[b], cls_v)
            pltpu.sync_copy(prior_hbm, prior_v)

            iota = lax.broadcasted_iota(jnp.int32, (16,), 0)
            zero_i = jnp.zeros((16,), jnp.int32)
            zero_f = jnp.zeros((16,), jnp.float32)
            for i in range(_NSLOT // 16):
                gcls_v[pl.ds(i * 16, 16)] = zero_i
                gox_v[pl.ds(i * 16, 16)] = zero_f
                goy_v[pl.ds(i * 16, 16)] = zero_f

            ks, cs, oxs, oys = [], [], [], []
            for c in range(4):
                ridx = jnp.minimum(iota + c * 16, _NOBJ - 1) * 4
                x0 = plsc.load_gather(box_v, [ridx])
                y0 = plsc.load_gather(box_v, [ridx + 1])
                x1 = plsc.load_gather(box_v, [ridx + 2])
                y1 = plsc.load_gather(box_v, [ridx + 3])
                cx = lax.shift_right_arithmetic(x0 + x1, 1)
                cy = lax.shift_right_arithmetic(y0 + y1, 1)
                kc = (lax.shift_right_arithmetic(cy, 5) * 10
                      + lax.shift_right_arithmetic(cx, 5))
                px = plsc.load_gather(prior_v, [kc * 2])
                py = plsc.load_gather(prior_v, [kc * 2 + 1])
                ks.append(kc)
                oxs.append(cx.astype(jnp.float32) - px)
                oys.append(cy.astype(jnp.float32) - py)
                cs.append(cls_v[pl.ds(c * 16, 16)])

            for n in range(_NOBJ):
                c, l = divmod(n, 16)
                m = iota == l
                plsc.store_scatter(gcls_v, [ks[c]], cs[c], mask=m)
                plsc.store_scatter(gox_v, [ks[c]], oxs[c], mask=m)
                plsc.store_scatter(goy_v, [ks[c]], oys[c], mask=m)

            pltpu.sync_copy(gcls_v, ocls.at[b, 0])
            pltpu.sync_copy(gox_v, oox.at[b, 0])
            pltpu.sync_copy(goy_v, ooy.at[b, 0])

    return k(boxes_p, cls_p, prior_p)


def _tc_loss(pred_r, gcls, gox, goy, interpret=False):
    B, CC, P = pred_r.shape
    NC = CC - 2

    def body(pred_ref, cls_ref, ox_ref, oy_ref, loc_ref, conf_ref, acc_ref):
        b = pl.program_id(0)
        x = pred_ref[0]
        rowi = lax.broadcasted_iota(jnp.int32, (CC, 1), 0)
        e = jnp.exp(x)
        ecls = jnp.where(rowi < NC, e, 0.0)
        sumexp = jnp.sum(ecls, axis=0, keepdims=True)
        lse = jnp.log(sumexp)
        s = x[0:1, :] - lse

        col = lax.broadcasted_iota(jnp.int32, (1, P), 1)
        m1 = jnp.min(s)
        i1 = jnp.min(jnp.where(s == m1, col, P))
        s2 = jnp.where(col == i1, jnp.float32(jnp.inf), s)
        m2 = jnp.min(s2)
        i2 = jnp.min(jnp.where(s2 == m2, col, P))

        g2 = cls_ref[0]
        pos = g2 > 0
        posf = jnp.where(pos, 1.0, 0.0)
        npos_b = jnp.sum(posf)

        xs = x[:, 0:_NSLOT]
        px = jnp.sum(jnp.where(rowi == NC, xs, 0.0), axis=0, keepdims=True)
        py = jnp.sum(jnp.where(rowi == NC + 1, xs, 0.0), axis=0, keepdims=True)
        gx = ox_ref[0]
        gy = oy_ref[0]
        loc_b = jnp.sum(((gx - px) ** 2 + (gy - py) ** 2) * posf)

        onehot = jnp.where(rowi == g2, 1.0, 0.0)
        picked = jnp.sum(xs * onehot, axis=0, keepdims=True)
        lse_s = lse[:, 0:_NSLOT]
        ce_b = jnp.sum((lse_s - picked) * posf)

        lane = lax.broadcasted_iota(jnp.int32, (1, _NSLOT), 1)
        isp1 = jnp.sum(jnp.where((lane == i1) & pos, 1.0, 0.0)) > 0.0
        isp2 = jnp.sum(jnp.where((lane == i2) & pos, 1.0, 0.0)) > 0.0
        ce_b = ce_b + jnp.where(isp1, 0.0, -m1) + jnp.where(isp2, 0.0, -m2)
        nsel_b = npos_b + jnp.where(isp1, 0.0, 1.0) + jnp.where(isp2, 0.0, 1.0)

        @pl.when(b == 0)
        def _():
            acc_ref[0] = loc_b
            acc_ref[1] = npos_b
            acc_ref[2] = ce_b
            acc_ref[3] = nsel_b

        @pl.when(b > 0)
        def _():
            acc_ref[0] += loc_b
            acc_ref[1] += npos_b
            acc_ref[2] += ce_b
            acc_ref[3] += nsel_b

        loc_ref[0, 0] = acc_ref[0] / (jnp.maximum(acc_ref[1], 1.0) * 2.0)
        conf_ref[0, 0] = acc_ref[2] / jnp.maximum(acc_ref[3], 1.0)

    return pl.pallas_call(
        body,
        grid=(B,),
        in_specs=[
            pl.BlockSpec((1, CC, P), lambda b: (b, 0, 0)),
            pl.BlockSpec((1, 1, _NSLOT), lambda b: (b, 0, 0)),
            pl.BlockSpec((1, 1, _NSLOT), lambda b: (b, 0, 0)),
            pl.BlockSpec((1, 1, _NSLOT), lambda b: (b, 0, 0)),
        ],
        out_specs=[
            pl.BlockSpec(memory_space=pltpu.SMEM),
            pl.BlockSpec(memory_space=pltpu.SMEM),
        ],
        out_shape=[
            jax.ShapeDtypeStruct((1, 1), jnp.float32),
            jax.ShapeDtypeStruct((1, 1), jnp.float32),
        ],
        scratch_shapes=[pltpu.SMEM((4,), jnp.float32)],
        interpret=interpret,
    )(pred_r, gcls, gox, goy)


def kernel(prediction_3d, boxes, classids, priorBox_2d):
    B = prediction_3d.shape[0]
    boxes_p = jnp.pad(boxes.astype(jnp.int32),
                      ((0, 0), (0, 6), (0, 0))).reshape(B, 224)
    cls_p = jnp.pad(classids.astype(jnp.int32), ((0, 0), (0, 14)))
    prior_p = priorBox_2d[:104].reshape(208)
    gcls, gox, goy = _sc_match(boxes_p, cls_p, prior_p)
    pred_r = jnp.concatenate(
        [prediction_3d[:, :, 2:], prediction_3d[:, :, :2]], axis=-1)
    pred_r = jnp.transpose(pred_r, (0, 2, 1))
    loc, conf = _tc_loss(pred_r, gcls, gox, goy)
    return (loc[0, 0], conf[0, 0])


# speedup vs baseline: 8.0362x; 1.0826x over previous
import functools

import jax
import jax.numpy as jnp
from jax import lax
from jax.experimental import pallas as pl
from jax.experimental.pallas import tpu as pltpu
from jax.experimental.pallas import tpu_sc as plsc

_NOBJ = 50
_NSLOT = 128


def _sc_match(boxes_p, cls_p, prior_p):
    B = boxes_p.shape[0]
    mesh = plsc.VectorSubcoreMesh(core_axis_name="c", subcore_axis_name="s")
    out_type = (
        jax.ShapeDtypeStruct((B, 1, _NSLOT), jnp.int32),
        jax.ShapeDtypeStruct((B, 1, _NSLOT), jnp.float32),
        jax.ShapeDtypeStruct((B, 1, _NSLOT), jnp.float32),
    )
    scratch = [
        pltpu.VMEM((224,), jnp.int32),
        pltpu.VMEM((64,), jnp.int32),
        pltpu.VMEM((208,), jnp.float32),
        pltpu.VMEM((_NSLOT,), jnp.int32),
        pltpu.VMEM((_NSLOT,), jnp.float32),
        pltpu.VMEM((_NSLOT,), jnp.float32),
    ]

    @functools.partial(
        pl.kernel, out_type=out_type, mesh=mesh, scratch_types=scratch,
        compiler_params=pltpu.CompilerParams(needs_layout_passes=False))
    def k(boxes_hbm, cls_hbm, prior_hbm, ocls, oox, ooy,
          box_v, cls_v, prior_v, gcls_v, gox_v, goy_v):
        wid = lax.axis_index("s") * 2 + lax.axis_index("c")

        @pl.when(wid < B)
        def _():
            b = wid
            pltpu.sync_copy(boxes_hbm.at[b], box_v)
            pltpu.sync_copy(cls_hbm.at[b], cls_v)
            pltpu.sync_copy(prior_hbm, prior_v)

            iota = lax.broadcasted_iota(jnp.int32, (16,), 0)
            zero_i = jnp.zeros((16,), jnp.int32)
            zero_f = jnp.zeros((16,), jnp.float32)
            for i in range(_NSLOT // 16):
                gcls_v[pl.ds(i * 16, 16)] = zero_i
                gox_v[pl.ds(i * 16, 16)] = zero_f
                goy_v[pl.ds(i * 16, 16)] = zero_f

            ks, cs, oxs, oys = [], [], [], []
            for c in range(4):
                ridx = jnp.minimum(iota + c * 16, _NOBJ - 1) * 4
                x0 = plsc.load_gather(box_v, [ridx])
                y0 = plsc.load_gather(box_v, [ridx + 1])
                x1 = plsc.load_gather(box_v, [ridx + 2])
                y1 = plsc.load_gather(box_v, [ridx + 3])
                cx = lax.shift_right_arithmetic(x0 + x1, 1)
                cy = lax.shift_right_arithmetic(y0 + y1, 1)
                kc = (lax.shift_right_arithmetic(cy, 5) * 10
                      + lax.shift_right_arithmetic(cx, 5))
                px = plsc.load_gather(prior_v, [kc * 2])
                py = plsc.load_gather(prior_v, [kc * 2 + 1])
                ks.append(kc)
                oxs.append(cx.astype(jnp.float32) - px)
                oys.append(cy.astype(jnp.float32) - py)
                cs.append(cls_v[pl.ds(c * 16, 16)])

            for n in range(_NOBJ):
                c, l = divmod(n, 16)
                m = iota == l
                plsc.store_scatter(gcls_v, [ks[c]], cs[c], mask=m)
                plsc.store_scatter(gox_v, [ks[c]], oxs[c], mask=m)
                plsc.store_scatter(goy_v, [ks[c]], oys[c], mask=m)

            pltpu.sync_copy(gcls_v, ocls.at[b, 0])
            pltpu.sync_copy(gox_v, oox.at[b, 0])
            pltpu.sync_copy(goy_v, ooy.at[b, 0])

    return k(boxes_p, cls_p, prior_p)


def _tc_loss(pred_r, pox, poy, gcls, gox, goy, interpret=False):
    B, NC, P = pred_r.shape

    def body(pred_ref, pox_ref, poy_ref, cls_ref, ox_ref, oy_ref,
             loc_ref, conf_ref, acc_ref):
        b = pl.program_id(0)
        x = pred_ref[0]
        e = jnp.exp(x)
        se = jnp.sum(e, axis=0, keepdims=True)
        r = e[0:1, :] / se

        col = lax.broadcasted_iota(jnp.int32, (1, P), 1)
        m1 = jnp.min(r)
        i1 = jnp.min(jnp.where(r == m1, col, P))
        r2 = jnp.where(col == i1, jnp.float32(jnp.inf), r)
        m2 = jnp.min(r2)
        i2 = jnp.min(jnp.where(r2 == m2, col, P))

        g2 = cls_ref[0]
        pos = g2 > 0
        posf = jnp.where(pos, 1.0, 0.0)
        npos_b = jnp.sum(posf)

        loc_b = jnp.sum(((ox_ref[0] - pox_ref[0]) ** 2
                         + (oy_ref[0] - poy_ref[0]) ** 2) * posf)

        xs = x[:, 0:_NSLOT]
        rowi = lax.broadcasted_iota(jnp.int32, (NC, 1), 0)
        onehot = jnp.where(rowi == g2, 1.0, 0.0)
        picked = jnp.sum(xs * onehot, axis=0, keepdims=True)
        lse_s = jnp.log(se[:, 0:_NSLOT])
        ce_b = jnp.sum((lse_s - picked) * posf)

        lane = lax.broadcasted_iota(jnp.int32, (1, _NSLOT), 1)
        isp1 = jnp.sum(jnp.where((lane == i1) & pos, 1.0, 0.0)) > 0.0
        isp2 = jnp.sum(jnp.where((lane == i2) & pos, 1.0, 0.0)) > 0.0
        ce_b = (ce_b + jnp.where(isp1, 0.0, -jnp.log(m1))
                + jnp.where(isp2, 0.0, -jnp.log(m2)))
        nsel_b = npos_b + jnp.where(isp1, 0.0, 1.0) + jnp.where(isp2, 0.0, 1.0)

        @pl.when(b == 0)
        def _():
            acc_ref[0] = loc_b
            acc_ref[1] = npos_b
            acc_ref[2] = ce_b
            acc_ref[3] = nsel_b

        @pl.when(b > 0)
        def _():
            acc_ref[0] += loc_b
            acc_ref[1] += npos_b
            acc_ref[2] += ce_b
            acc_ref[3] += nsel_b

        loc_ref[0, 0] = acc_ref[0] / (jnp.maximum(acc_ref[1], 1.0) * 2.0)
        conf_ref[0, 0] = acc_ref[2] / jnp.maximum(acc_ref[3], 1.0)

    return pl.pallas_call(
        body,
        grid=(B,),
        in_specs=[
            pl.BlockSpec((1, NC, P), lambda b: (b, 0, 0)),
            pl.BlockSpec((1, 1, _NSLOT), lambda b: (b, 0, 0)),
            pl.BlockSpec((1, 1, _NSLOT), lambda b: (b, 0, 0)),
            pl.BlockSpec((1, 1, _NSLOT), lambda b: (b, 0, 0)),
            pl.BlockSpec((1, 1, _NSLOT), lambda b: (b, 0, 0)),
            pl.BlockSpec((1, 1, _NSLOT), lambda b: (b, 0, 0)),
        ],
        out_specs=[
            pl.BlockSpec(memory_space=pltpu.SMEM),
            pl.BlockSpec(memory_space=pltpu.SMEM),
        ],
        out_shape=[
            jax.ShapeDtypeStruct((1, 1), jnp.float32),
            jax.ShapeDtypeStruct((1, 1), jnp.float32),
        ],
        scratch_shapes=[pltpu.SMEM((4,), jnp.float32)],
        interpret=interpret,
    )(pred_r, pox, poy, gcls, gox, goy)


def kernel(prediction_3d, boxes, classids, priorBox_2d):
    B = prediction_3d.shape[0]
    boxes_p = jnp.pad(boxes.astype(jnp.int32),
                      ((0, 0), (0, 6), (0, 0))).reshape(B, 224)
    cls_p = jnp.pad(classids.astype(jnp.int32), ((0, 0), (0, 14)))
    prior_p = priorBox_2d[:104].reshape(208)
    gcls, gox, goy = _sc_match(boxes_p, cls_p, prior_p)
    pred_r = jnp.transpose(prediction_3d[:, :, 2:], (0, 2, 1))
    po = prediction_3d[:, :_NSLOT, :2]
    pox = po[:, :, 0].reshape(B, 1, _NSLOT)
    poy = po[:, :, 1].reshape(B, 1, _NSLOT)
    loc, conf = _tc_loss(pred_r, pox, poy, gcls, gox, goy)
    return (loc[0, 0], conf[0, 0])


# speedup vs baseline: 8.1824x; 1.0182x over previous
import functools

import jax
import jax.numpy as jnp
from jax import lax
from jax.experimental import pallas as pl
from jax.experimental.pallas import tpu as pltpu
from jax.experimental.pallas import tpu_sc as plsc

_NOBJ = 50
_NSLOT = 128


def _sc_match(boxes_p, cls_p, prior_p):
    B = boxes_p.shape[0]
    mesh = plsc.VectorSubcoreMesh(core_axis_name="c", subcore_axis_name="s")
    out_type = (
        jax.ShapeDtypeStruct((B, 1, _NSLOT), jnp.int32),
        jax.ShapeDtypeStruct((B, 1, _NSLOT), jnp.float32),
        jax.ShapeDtypeStruct((B, 1, _NSLOT), jnp.float32),
    )
    scratch = [
        pltpu.VMEM((224,), jnp.int32),
        pltpu.VMEM((64,), jnp.int32),
        pltpu.VMEM((208,), jnp.float32),
        pltpu.VMEM((_NSLOT,), jnp.int32),
        pltpu.VMEM((_NSLOT,), jnp.float32),
        pltpu.VMEM((_NSLOT,), jnp.float32),
    ]

    @functools.partial(
        pl.kernel, out_type=out_type, mesh=mesh, scratch_types=scratch,
        compiler_params=pltpu.CompilerParams(needs_layout_passes=False))
    def k(boxes_hbm, cls_hbm, prior_hbm, ocls, oox, ooy,
          box_v, cls_v, prior_v, gcls_v, gox_v, goy_v):
        wid = lax.axis_index("s") * 2 + lax.axis_index("c")

        @pl.when(wid < B)
        def _():
            b = wid
            pltpu.sync_copy(boxes_hbm.at[b], box_v)
            pltpu.sync_copy(cls_hbm.at[b], cls_v)
            pltpu.sync_copy(prior_hbm, prior_v)

            iota = lax.broadcasted_iota(jnp.int32, (16,), 0)
            zero_i = jnp.zeros((16,), jnp.int32)
            zero_f = jnp.zeros((16,), jnp.float32)
            for i in range(_NSLOT // 16):
                gcls_v[pl.ds(i * 16, 16)] = zero_i
                gox_v[pl.ds(i * 16, 16)] = zero_f
                goy_v[pl.ds(i * 16, 16)] = zero_f

            ks, cs, oxs, oys = [], [], [], []
            for c in range(4):
                ridx = jnp.minimum(iota + c * 16, _NOBJ - 1) * 4
                x0 = plsc.load_gather(box_v, [ridx])
                y0 = plsc.load_gather(box_v, [ridx + 1])
                x1 = plsc.load_gather(box_v, [ridx + 2])
                y1 = plsc.load_gather(box_v, [ridx + 3])
                cx = lax.shift_right_arithmetic(x0 + x1, 1)
                cy = lax.shift_right_arithmetic(y0 + y1, 1)
                kc = (lax.shift_right_arithmetic(cy, 5) * 10
                      + lax.shift_right_arithmetic(cx, 5))
                px = plsc.load_gather(prior_v, [kc * 2])
                py = plsc.load_gather(prior_v, [kc * 2 + 1])
                ks.append(kc)
                oxs.append(cx.astype(jnp.float32) - px)
                oys.append(cy.astype(jnp.float32) - py)
                cs.append(cls_v[pl.ds(c * 16, 16)])

            for n in range(_NOBJ):
                c, l = divmod(n, 16)
                m = iota == l
                plsc.store_scatter(gcls_v, [ks[c]], cs[c], mask=m)
                plsc.store_scatter(gox_v, [ks[c]], oxs[c], mask=m)
                plsc.store_scatter(goy_v, [ks[c]], oys[c], mask=m)

            pltpu.sync_copy(gcls_v, ocls.at[b, 0])
            pltpu.sync_copy(gox_v, oox.at[b, 0])
            pltpu.sync_copy(goy_v, ooy.at[b, 0])

    return k(boxes_p, cls_p, prior_p)


def _tc_loss(pred_r, pox, poy, gcls, gox, goy, interpret=False):
    B, NC, P = pred_r.shape

    def body(pred_ref, pox_ref, poy_ref, cls_ref, ox_ref, oy_ref,
             loc_ref, conf_ref, acc_ref):
        b = pl.program_id(0)
        x = pred_ref[0]
        e = jnp.exp(x)
        se = jax.lax.dot_general(
            jnp.ones((1, NC), jnp.float32), e, (((1,), (0,)), ((), ())),
            preferred_element_type=jnp.float32)
        r = e[0:1, :] / se

        col = lax.broadcasted_iota(jnp.int32, (1, P), 1)
        m1 = jnp.min(r)
        i1 = jnp.min(jnp.where(r == m1, col, P))
        r2 = jnp.where(col == i1, jnp.float32(jnp.inf), r)
        m2 = jnp.min(r2)
        i2 = jnp.min(jnp.where(r2 == m2, col, P))

        g2 = cls_ref[0]
        pos = g2 > 0
        posf = jnp.where(pos, 1.0, 0.0)
        npos_b = jnp.sum(posf)

        loc_b = jnp.sum(((ox_ref[0] - pox_ref[0]) ** 2
                         + (oy_ref[0] - poy_ref[0]) ** 2) * posf)

        xs = x[:, 0:_NSLOT]
        rowi = lax.broadcasted_iota(jnp.int32, (NC, 1), 0)
        onehot = jnp.where(rowi == g2, 1.0, 0.0)
        picked = jnp.sum(xs * onehot, axis=0, keepdims=True)
        lse_s = jnp.log(se[:, 0:_NSLOT])
        ce_b = jnp.sum((lse_s - picked) * posf)

        lane = lax.broadcasted_iota(jnp.int32, (1, _NSLOT), 1)
        isp1 = jnp.sum(jnp.where((lane == i1) & pos, 1.0, 0.0)) > 0.0
        isp2 = jnp.sum(jnp.where((lane == i2) & pos, 1.0, 0.0)) > 0.0
        ce_b = (ce_b + jnp.where(isp1, 0.0, -jnp.log(m1))
                + jnp.where(isp2, 0.0, -jnp.log(m2)))
        nsel_b = npos_b + jnp.where(isp1, 0.0, 1.0) + jnp.where(isp2, 0.0, 1.0)

        @pl.when(b == 0)
        def _():
            acc_ref[0] = loc_b
            acc_ref[1] = npos_b
            acc_ref[2] = ce_b
            acc_ref[3] = nsel_b

        @pl.when(b > 0)
        def _():
            acc_ref[0] += loc_b
            acc_ref[1] += npos_b
            acc_ref[2] += ce_b
            acc_ref[3] += nsel_b

        loc_ref[0, 0] = acc_ref[0] / (jnp.maximum(acc_ref[1], 1.0) * 2.0)
        conf_ref[0, 0] = acc_ref[2] / jnp.maximum(acc_ref[3], 1.0)

    return pl.pallas_call(
        body,
        grid=(B,),
        in_specs=[
            pl.BlockSpec((1, NC, P), lambda b: (b, 0, 0)),
            pl.BlockSpec((1, 1, _NSLOT), lambda b: (b, 0, 0)),
            pl.BlockSpec((1, 1, _NSLOT), lambda b: (b, 0, 0)),
            pl.BlockSpec((1, 1, _NSLOT), lambda b: (b, 0, 0)),
            pl.BlockSpec((1, 1, _NSLOT), lambda b: (b, 0, 0)),
            pl.BlockSpec((1, 1, _NSLOT), lambda b: (b, 0, 0)),
        ],
        out_specs=[
            pl.BlockSpec(memory_space=pltpu.SMEM),
            pl.BlockSpec(memory_space=pltpu.SMEM),
        ],
        out_shape=[
            jax.ShapeDtypeStruct((1, 1), jnp.float32),
            jax.ShapeDtypeStruct((1, 1), jnp.float32),
        ],
        scratch_shapes=[pltpu.SMEM((4,), jnp.float32)],
        interpret=interpret,
    )(pred_r, pox, poy, gcls, gox, goy)


def kernel(prediction_3d, boxes, classids, priorBox_2d):
    B = prediction_3d.shape[0]
    boxes_p = jnp.pad(boxes.astype(jnp.int32),
                      ((0, 0), (0, 6), (0, 0))).reshape(B, 224)
    cls_p = jnp.pad(classids.astype(jnp.int32), ((0, 0), (0, 14)))
    prior_p = priorBox_2d[:104].reshape(208)
    gcls, gox, goy = _sc_match(boxes_p, cls_p, prior_p)
    pred_r = jnp.transpose(prediction_3d[:, :, 2:], (0, 2, 1))
    po = prediction_3d[:, :_NSLOT, :2]
    pox = po[:, :, 0].reshape(B, 1, _NSLOT)
    poy = po[:, :, 1].reshape(B, 1, _NSLOT)
    loc, conf = _tc_loss(pred_r, pox, poy, gcls, gox, goy)
    return (loc[0, 0], conf[0, 0])
